# baseline (speedup 1.0000x reference)
import functools

import jax
import jax.numpy as jnp
from jax import lax
from jax.experimental import pallas as pl
from jax.experimental.pallas import tpu as pltpu
from jax.experimental.pallas import tpu_sc as plsc

N_CORES = 2
N_SUBCORES = 16
NW = N_CORES * N_SUBCORES
KB = 128
NB = 4
N_ZT = 10


def _mesh():
    return plsc.VectorSubcoreMesh(core_axis_name="c", subcore_axis_name="s")


def _make_sc_degree(n_nodes, n_acc, rows_pt):
    zpt = n_acc // N_ZT
    rpt = n_nodes // N_ZT

    @functools.partial(
        pl.kernel,
        out_type=jax.ShapeDtypeStruct((N_CORES, n_nodes, 8), jnp.float32),
        mesh=_mesh(),
        compiler_params=pltpu.CompilerParams(use_tc_tiling_on_sc=False),
        scratch_types=[
            pltpu.VMEM((rows_pt, KB), jnp.int32),
            pltpu.VMEM((KB, 8), jnp.float32),
            pltpu.VMEM_SHARED((n_acc, 8), jnp.float32),
            pltpu.SemaphoreType.DMA,
        ],
    )
    def deg_kernel(edges_hbm, zeros_hbm, ones_hbm, out_hbm,
                   idx_v, ones_v, deg_sh, dsem):
        c = lax.axis_index("c")
        s = lax.axis_index("s")
        wid = c * N_SUBCORES + s

        @pl.when(s < N_ZT)
        def _zero():
            pltpu.sync_copy(zeros_hbm.at[pl.ds(s * zpt, zpt)],
                            deg_sh.at[pl.ds(s * zpt, zpt)])

        pltpu.sync_copy(ones_hbm, ones_v)
        pltpu.sync_copy(edges_hbm.at[1, wid], idx_v)
        plsc.subcore_barrier()

        def fire(j, carry):
            pltpu.async_copy(ones_v, deg_sh.at[idx_v.at[j]], dsem, add=True)
            return carry

        lax.fori_loop(0, rows_pt, fire, 0)

        def drain(j, carry):
            pltpu.make_async_copy(ones_v, deg_sh.at[idx_v.at[j]], dsem).wait()
            return carry

        lax.fori_loop(0, rows_pt, drain, 0)
        plsc.subcore_barrier()

        @pl.when(s < N_ZT)
        def _out():
            pltpu.sync_copy(deg_sh.at[pl.ds(s * rpt, rpt)],
                            out_hbm.at[c, pl.ds(s * rpt, rpt)])

    return deg_kernel


def _make_sc_scatter(n_nodes, n_acc, rows_pt, d):
    n_grp = rows_pt // NB
    zpt = n_acc // N_ZT
    rpt = n_nodes // N_ZT
    ns = 2 if d >= 64 else 3

    @functools.partial(
        pl.kernel,
        out_type=jax.ShapeDtypeStruct((N_CORES, n_nodes, d), jnp.float32),
        mesh=_mesh(),
        compiler_params=pltpu.CompilerParams(use_tc_tiling_on_sc=False),
        scratch_types=[
            pltpu.VMEM((rows_pt, KB), jnp.int32),
            pltpu.VMEM((rows_pt, KB), jnp.int32),
            pltpu.VMEM((ns, NB, KB, d), jnp.float32),
            pltpu.VMEM_SHARED((n_acc, d), jnp.float32),
            pltpu.SemaphoreType.DMA((ns, NB)),
            pltpu.SemaphoreType.DMA((ns, NB)),
        ],
    )
    def scatter_kernel(edges_hbm, feat_hbm, zeros_hbm, out_hbm,
                       sidx_v, didx_v, rows_v, acc_sh, gsem, ssem):
        c = lax.axis_index("c")
        s = lax.axis_index("s")
        wid = c * N_SUBCORES + s

        @pl.when(s < N_ZT)
        def _zero():
            pltpu.sync_copy(zeros_hbm.at[pl.ds(s * zpt, zpt)],
                            acc_sh.at[pl.ds(s * zpt, zpt)])

        pltpu.sync_copy(edges_hbm.at[0, wid], sidx_v)
        pltpu.sync_copy(edges_hbm.at[1, wid], didx_v)
        plsc.subcore_barrier()

        def g_issue(p, b, j):
            pltpu.async_copy(feat_hbm.at[sidx_v.at[j]], rows_v.at[p, b],
                             gsem.at[p, b])

        def g_wait(p, b, j):
            pltpu.make_async_copy(feat_hbm.at[sidx_v.at[j]], rows_v.at[p, b],
                                  gsem.at[p, b]).wait()

        def s_issue(p, b, j):
            pltpu.async_copy(rows_v.at[p, b], acc_sh.at[didx_v.at[j]],
                             ssem.at[p, b], add=True)

        def s_wait(p, b, j):
            pltpu.make_async_copy(rows_v.at[p, b], acc_sh.at[didx_v.at[j]],
                                  ssem.at[p, b]).wait()

        def step(g, p, do_swait=True, do_gissue=True):
            base = g * NB
            for b in range(NB):
                g_wait(p, b, base + b)
                s_issue(p, b, base + b)
            q = (p + ns - 1) % ns
            for b in range(NB):
                if do_swait:
                    s_wait(q, b, base - NB + b)
                if do_gissue:
                    g_issue(q, b, base + (ns - 1) * NB + b)

        for g0 in range(ns - 1):
            for b in range(NB):
                g_issue(g0, b, g0 * NB + b)
        step(0, 0, do_swait=False)

        n_steady = n_grp - ns
        n_iter = n_steady // ns

        def outer(gi, carry):
            g = ns * gi + 1
            for r in range(ns):
                step(g + r, (1 + r) % ns)
            return carry

        lax.fori_loop(0, n_iter, outer, 0)
        for g in range(n_iter * ns + 1, n_grp):
            step(g, g % ns, do_gissue=(g + ns - 1 < n_grp))
        for b in range(NB):
            s_wait((n_grp - 1) % ns, b, (n_grp - 1) * NB + b)
        plsc.subcore_barrier()

        @pl.when(s < N_ZT)
        def _out():
            pltpu.sync_copy(acc_sh.at[pl.ds(s * rpt, rpt)],
                            out_hbm.at[c, pl.ds(s * rpt, rpt)])

    return scatter_kernel


def _tc_layer0(x, w0, b0, degp, *, rows_blk=2000):
    n, d_in = x.shape
    d_hid = w0.shape[1]
    grid = (n // rows_blk,)

    def body(x_ref, w0_ref, b0_ref, degp_ref, hs_ref, dinv_ref):
        deg = degp_ref[0, :, 0:1] + degp_ref[1, :, 0:1] + 1.0
        dinv = lax.rsqrt(deg)
        h = jnp.dot(x_ref[...], w0_ref[...],
                    preferred_element_type=jnp.float32) + b0_ref[...]
        hs_ref[...] = h * dinv
        dinv_ref[...] = dinv

    return pl.pallas_call(
        body,
        grid=grid,
        in_specs=[
            pl.BlockSpec((rows_blk, d_in), lambda i: (i, 0)),
            pl.BlockSpec((d_in, d_hid), lambda i: (0, 0)),
            pl.BlockSpec((1, d_hid), lambda i: (0, 0)),
            pl.BlockSpec((N_CORES, rows_blk, 8), lambda i: (0, i, 0)),
        ],
        out_specs=[
            pl.BlockSpec((rows_blk, d_hid), lambda i: (i, 0)),
            pl.BlockSpec((rows_blk, 1), lambda i: (i, 0)),
        ],
        out_shape=[
            jax.ShapeDtypeStruct((n, d_hid), jnp.float32),
            jax.ShapeDtypeStruct((n, 1), jnp.float32),
        ],
    )(x, w0, b0, degp)


def _tc_layer1(accp, hs, dinv, w1, b1, *, rows_blk=2000):
    n, d_hid = hs.shape
    d_out = w1.shape[1]
    grid = (n // rows_blk,)

    def body(accp_ref, hs_ref, dinv_ref, w1_ref, b1_ref, os_ref):
        dinv = dinv_ref[...]
        sm = dinv * (accp_ref[0] + accp_ref[1] + hs_ref[...])
        h1 = jnp.maximum(sm, 0.0)
        os_ref[...] = (jnp.dot(h1, w1_ref[...],
                               preferred_element_type=jnp.float32)
                       + b1_ref[...]) * dinv

    return pl.pallas_call(
        body,
        grid=grid,
        in_specs=[
            pl.BlockSpec((N_CORES, rows_blk, d_hid), lambda i: (0, i, 0)),
            pl.BlockSpec((rows_blk, d_hid), lambda i: (i, 0)),
            pl.BlockSpec((rows_blk, 1), lambda i: (i, 0)),
            pl.BlockSpec((d_hid, d_out), lambda i: (0, 0)),
            pl.BlockSpec((1, d_out), lambda i: (0, 0)),
        ],
        out_specs=pl.BlockSpec((rows_blk, d_out), lambda i: (i, 0)),
        out_shape=jax.ShapeDtypeStruct((n, d_out), jnp.float32),
    )(accp, hs, dinv, w1, b1)


def _tc_final(accp2, os_, dinv, *, rows_blk=2000):
    n, d_out = os_.shape
    grid = (n // rows_blk,)

    def body(accp_ref, os_ref, dinv_ref, o_ref):
        o_ref[...] = dinv_ref[...] * (accp_ref[0] + accp_ref[1] + os_ref[...])

    return pl.pallas_call(
        body,
        grid=grid,
        in_specs=[
            pl.BlockSpec((N_CORES, rows_blk, d_out), lambda i: (0, i, 0)),
            pl.BlockSpec((rows_blk, d_out), lambda i: (i, 0)),
            pl.BlockSpec((rows_blk, 1), lambda i: (i, 0)),
        ],
        out_specs=pl.BlockSpec((rows_blk, d_out), lambda i: (i, 0)),
        out_shape=jax.ShapeDtypeStruct((n, d_out), jnp.float32),
    )(accp2, os_, dinv)


def kernel(X, edge_index, W0, b0, W1, b1):
    n, d_in = X.shape
    d_hid = W0.shape[1]
    d_out = W1.shape[1]
    e = edge_index.shape[1]

    chunk = NW * KB * NB
    e_pad = ((e + chunk - 1) // chunk) * chunk
    rows_pt = e_pad // (NW * KB)
    pad_n = e_pad - e
    src_p = jnp.concatenate([edge_index[0], jnp.zeros((pad_n,), jnp.int32)])
    dst_p = jnp.concatenate([edge_index[1], jnp.full((pad_n,), n, jnp.int32)])
    edges4d = jnp.stack([src_p, dst_p]).reshape(2, NW, rows_pt, KB)
    n_acc = ((n + 1 + 8 * N_ZT - 1) // (8 * N_ZT)) * (8 * N_ZT)
    b0r = b0.reshape(1, d_hid)
    b1r = b1.reshape(1, d_out)

    zeros_deg = jnp.zeros((n_acc, 8), jnp.float32)
    ones8 = jnp.ones((KB, 8), jnp.float32)
    zeros_hid = jnp.zeros((n_acc, d_hid), jnp.float32)
    zeros_out = jnp.zeros((n_acc, d_out), jnp.float32)

    degp = _make_sc_degree(n, n_acc, rows_pt)(edges4d, zeros_deg, ones8)
    hs, dinv = _tc_layer0(X, W0, b0r, degp)
    accp = _make_sc_scatter(n, n_acc, rows_pt, d_hid)(edges4d, hs, zeros_hid)
    os_ = _tc_layer1(accp, hs, dinv, W1, b1r)
    accp2 = _make_sc_scatter(n, n_acc, rows_pt, d_out)(edges4d, os_, zeros_out)
    return _tc_final(accp2, os_, dinv)


# speedup vs baseline: 1.0174x; 1.0174x over previous
import functools

import jax
import jax.numpy as jnp
from jax import lax
from jax.experimental import pallas as pl
from jax.experimental.pallas import tpu as pltpu
from jax.experimental.pallas import tpu_sc as plsc

N_CORES = 2
N_SUBCORES = 16
NW = N_CORES * N_SUBCORES
KB = 128
NB = 4
N_ZT = 10


def _mesh():
    return plsc.VectorSubcoreMesh(core_axis_name="c", subcore_axis_name="s")


def _make_sc_degree(n_nodes, n_acc, rows_pt):
    zpt = n_acc // N_ZT
    rpt = n_nodes // N_ZT

    @functools.partial(
        pl.kernel,
        out_type=jax.ShapeDtypeStruct((N_CORES, n_nodes, 8), jnp.float32),
        mesh=_mesh(),
        compiler_params=pltpu.CompilerParams(use_tc_tiling_on_sc=False),
        scratch_types=[
            pltpu.VMEM((rows_pt, KB), jnp.int32),
            pltpu.VMEM((KB, 8), jnp.float32),
            pltpu.VMEM_SHARED((n_acc, 8), jnp.float32),
            pltpu.SemaphoreType.DMA,
        ],
    )
    def deg_kernel(edges_hbm, zeros_hbm, ones_hbm, out_hbm,
                   idx_v, ones_v, deg_sh, dsem):
        c = lax.axis_index("c")
        s = lax.axis_index("s")
        wid = c * N_SUBCORES + s

        @pl.when(s < N_ZT)
        def _zero():
            pltpu.sync_copy(zeros_hbm.at[pl.ds(s * zpt, zpt)],
                            deg_sh.at[pl.ds(s * zpt, zpt)])

        pltpu.sync_copy(ones_hbm, ones_v)
        pltpu.sync_copy(edges_hbm.at[1, wid], idx_v)
        plsc.subcore_barrier()

        def fire(j, carry):
            pltpu.async_copy(ones_v, deg_sh.at[idx_v.at[j]], dsem, add=True)
            return carry

        lax.fori_loop(0, rows_pt, fire, 0)

        def drain(j, carry):
            pltpu.make_async_copy(ones_v, deg_sh.at[idx_v.at[j]], dsem).wait()
            return carry

        lax.fori_loop(0, rows_pt, drain, 0)
        plsc.subcore_barrier()

        @pl.when(s < N_ZT)
        def _out():
            pltpu.sync_copy(deg_sh.at[pl.ds(s * rpt, rpt)],
                            out_hbm.at[c, pl.ds(s * rpt, rpt)])

    return deg_kernel


def _make_sc_scatter(n_nodes, n_acc, rows_pt, d):
    n_grp = rows_pt // NB
    zpt = n_acc // N_ZT
    rpt = n_nodes // N_ZT
    ns = 2 if d >= 64 else 3

    @functools.partial(
        pl.kernel,
        out_type=jax.ShapeDtypeStruct((N_CORES, n_nodes, d), jnp.float32),
        mesh=_mesh(),
        compiler_params=pltpu.CompilerParams(use_tc_tiling_on_sc=False),
        scratch_types=[
            pltpu.VMEM((rows_pt, KB), jnp.int32),
            pltpu.VMEM((rows_pt, KB), jnp.int32),
            pltpu.VMEM((ns, NB, KB, d), jnp.float32),
            pltpu.VMEM_SHARED((n_acc, d), jnp.float32),
            pltpu.SemaphoreType.DMA((ns, NB)),
            pltpu.SemaphoreType.DMA((ns, NB)),
        ],
    )
    def scatter_kernel(edges_hbm, feat_hbm, zeros_hbm, out_hbm,
                       sidx_v, didx_v, rows_v, acc_sh, gsem, ssem):
        c = lax.axis_index("c")
        s = lax.axis_index("s")
        wid = c * N_SUBCORES + s

        @pl.when(s < N_ZT)
        def _zero():
            pltpu.sync_copy(zeros_hbm.at[pl.ds(s * zpt, zpt)],
                            acc_sh.at[pl.ds(s * zpt, zpt)])

        pltpu.sync_copy(edges_hbm.at[0, wid], sidx_v)
        pltpu.sync_copy(edges_hbm.at[1, wid], didx_v)
        plsc.subcore_barrier()

        def g_issue(p, b, j):
            pltpu.async_copy(feat_hbm.at[sidx_v.at[j]], rows_v.at[p, b],
                             gsem.at[p, b])

        def g_wait(p, b, j):
            pltpu.make_async_copy(feat_hbm.at[sidx_v.at[j]], rows_v.at[p, b],
                                  gsem.at[p, b]).wait()

        def s_issue(p, b, j):
            pltpu.async_copy(rows_v.at[p, b], acc_sh.at[didx_v.at[j]],
                             ssem.at[p, b], add=True)

        def s_wait(p, b, j):
            pltpu.make_async_copy(rows_v.at[p, b], acc_sh.at[didx_v.at[j]],
                                  ssem.at[p, b]).wait()

        def step(g, p, do_swait=True, do_gissue=True):
            base = g * NB
            for b in range(NB):
                g_wait(p, b, base + b)
                s_issue(p, b, base + b)
            q = (p + ns - 1) % ns
            for b in range(NB):
                if do_swait:
                    s_wait(q, b, base - NB + b)
                if do_gissue:
                    g_issue(q, b, base + (ns - 1) * NB + b)

        for g0 in range(ns - 1):
            for b in range(NB):
                g_issue(g0, b, g0 * NB + b)
        step(0, 0, do_swait=False)

        n_steady = n_grp - ns
        n_iter = n_steady // ns

        def outer(gi, carry):
            g = ns * gi + 1
            for r in range(ns):
                step(g + r, (1 + r) % ns)
            return carry

        lax.fori_loop(0, n_iter, outer, 0)
        for g in range(n_iter * ns + 1, n_grp):
            step(g, g % ns, do_gissue=(g + ns - 1 < n_grp))
        for b in range(NB):
            s_wait((n_grp - 1) % ns, b, (n_grp - 1) * NB + b)
        plsc.subcore_barrier()

        @pl.when(s < N_ZT)
        def _out():
            pltpu.sync_copy(acc_sh.at[pl.ds(s * rpt, rpt)],
                            out_hbm.at[c, pl.ds(s * rpt, rpt)])

    return scatter_kernel


def _tc_layer0(x, w0, b0, degp, *, rows_blk=2000):
    n, d_in = x.shape
    d_hid = w0.shape[1]
    grid = (n // rows_blk,)

    def body(x_ref, w0_ref, b0_ref, degp_ref, hs_ref, dinv_ref):
        deg = degp_ref[0, :, 0:1] + degp_ref[1, :, 0:1] + 1.0
        dinv = lax.rsqrt(deg)
        h = jnp.dot(x_ref[...], w0_ref[...],
                    preferred_element_type=jnp.float32) + b0_ref[...]
        hs_ref[...] = h * dinv
        dinv_ref[...] = dinv

    return pl.pallas_call(
        body,
        grid=grid,
        in_specs=[
            pl.BlockSpec((rows_blk, d_in), lambda i: (i, 0)),
            pl.BlockSpec((d_in, d_hid), lambda i: (0, 0)),
            pl.BlockSpec((1, d_hid), lambda i: (0, 0)),
            pl.BlockSpec((N_CORES, rows_blk, 8), lambda i: (0, i, 0)),
        ],
        out_specs=[
            pl.BlockSpec((rows_blk, d_hid), lambda i: (i, 0)),
            pl.BlockSpec((rows_blk, 1), lambda i: (i, 0)),
        ],
        out_shape=[
            jax.ShapeDtypeStruct((n, d_hid), jnp.float32),
            jax.ShapeDtypeStruct((n, 1), jnp.float32),
        ],
    )(x, w0, b0, degp)


def _tc_layer1(accp, hs, dinv, w1, b1, *, rows_blk=2000):
    n, d_hid = hs.shape
    d_out = w1.shape[1]
    grid = (n // rows_blk,)

    def body(accp_ref, hs_ref, dinv_ref, w1_ref, b1_ref, os_ref):
        dinv = dinv_ref[...]
        sm = dinv * (accp_ref[0] + accp_ref[1] + hs_ref[...])
        h1 = jnp.maximum(sm, 0.0)
        os_ref[...] = (jnp.dot(h1, w1_ref[...],
                               preferred_element_type=jnp.float32)
                       + b1_ref[...]) * dinv

    return pl.pallas_call(
        body,
        grid=grid,
        in_specs=[
            pl.BlockSpec((N_CORES, rows_blk, d_hid), lambda i: (0, i, 0)),
            pl.BlockSpec((rows_blk, d_hid), lambda i: (i, 0)),
            pl.BlockSpec((rows_blk, 1), lambda i: (i, 0)),
            pl.BlockSpec((d_hid, d_out), lambda i: (0, 0)),
            pl.BlockSpec((1, d_out), lambda i: (0, 0)),
        ],
        out_specs=pl.BlockSpec((rows_blk, d_out), lambda i: (i, 0)),
        out_shape=jax.ShapeDtypeStruct((n, d_out), jnp.float32),
    )(accp, hs, dinv, w1, b1)


def _tc_final(accp2, os_, dinv, *, rows_blk=2000):
    n, d_out = os_.shape
    grid = (n // rows_blk,)

    def body(accp_ref, os_ref, dinv_ref, o_ref):
        o_ref[...] = dinv_ref[...] * (accp_ref[0] + accp_ref[1] + os_ref[...])

    return pl.pallas_call(
        body,
        grid=grid,
        in_specs=[
            pl.BlockSpec((N_CORES, rows_blk, d_out), lambda i: (0, i, 0)),
            pl.BlockSpec((rows_blk, d_out), lambda i: (i, 0)),
            pl.BlockSpec((rows_blk, 1), lambda i: (i, 0)),
        ],
        out_specs=pl.BlockSpec((rows_blk, d_out), lambda i: (i, 0)),
        out_shape=jax.ShapeDtypeStruct((n, d_out), jnp.float32),
    )(accp2, os_, dinv)


def kernel(X, edge_index, W0, b0, W1, b1):
    n, d_in = X.shape
    d_hid = W0.shape[1]
    d_out = W1.shape[1]
    e = edge_index.shape[1]

    chunk = NW * KB * NB
    e_pad = ((e + chunk - 1) // chunk) * chunk
    rows_pt = e_pad // (NW * KB)
    pad_n = e_pad - e
    n_acc = ((n + 1 + 8 * N_ZT - 1) // (8 * N_ZT)) * (8 * N_ZT)
    trash = n + jnp.arange(pad_n, dtype=jnp.int32) % jnp.int32(n_acc - n)
    src_p = jnp.concatenate([edge_index[0], jnp.zeros((pad_n,), jnp.int32)])
    dst_p = jnp.concatenate([edge_index[1], trash])
    edges4d = jnp.stack([src_p, dst_p]).reshape(2, NW, rows_pt, KB)
    b0r = b0.reshape(1, d_hid)
    b1r = b1.reshape(1, d_out)

    zeros_deg = jnp.zeros((n_acc, 8), jnp.float32)
    ones8 = jnp.ones((KB, 8), jnp.float32)
    zeros_hid = jnp.zeros((n_acc, d_hid), jnp.float32)
    zeros_out = jnp.zeros((n_acc, d_out), jnp.float32)

    degp = _make_sc_degree(n, n_acc, rows_pt)(edges4d, zeros_deg, ones8)
    hs, dinv = _tc_layer0(X, W0, b0r, degp)
    accp = _make_sc_scatter(n, n_acc, rows_pt, d_hid)(edges4d, hs, zeros_hid)
    os_ = _tc_layer1(accp, hs, dinv, W1, b1r)
    accp2 = _make_sc_scatter(n, n_acc, rows_pt, d_out)(edges4d, os_, zeros_out)
    return _tc_final(accp2, os_, dinv)


# speedup vs baseline: 2.3336x; 2.2937x over previous
import functools

import jax
import jax.numpy as jnp
from jax import lax
from jax.experimental import pallas as pl
from jax.experimental.pallas import tpu as pltpu
from jax.experimental.pallas import tpu_sc as plsc

N_CORES = 2
N_SUBCORES = 16
NW = N_CORES * N_SUBCORES
KB = 80
NB = 5
N_ZT = 10


def _mesh():
    return plsc.VectorSubcoreMesh(core_axis_name="c", subcore_axis_name="s")


def _make_sc_degree(n_nodes, n_acc, rows_pt):
    zpt = n_acc // N_ZT
    rpt = n_nodes // N_ZT

    @functools.partial(
        pl.kernel,
        out_type=jax.ShapeDtypeStruct((N_CORES, n_nodes, 8), jnp.float32),
        mesh=_mesh(),
        compiler_params=pltpu.CompilerParams(use_tc_tiling_on_sc=False),
        scratch_types=[
            pltpu.VMEM((rows_pt, KB), jnp.int32),
            pltpu.VMEM((KB, 8), jnp.float32),
            pltpu.VMEM_SHARED((n_acc, 8), jnp.float32),
            pltpu.SemaphoreType.DMA,
        ],
    )
    def deg_kernel(edges_hbm, zeros_hbm, ones_hbm, out_hbm,
                   idx_v, ones_v, deg_sh, dsem):
        c = lax.axis_index("c")
        s = lax.axis_index("s")
        wid = c * N_SUBCORES + s

        @pl.when(s < N_ZT)
        def _zero():
            pltpu.sync_copy(zeros_hbm.at[pl.ds(s * zpt, zpt)],
                            deg_sh.at[pl.ds(s * zpt, zpt)])

        pltpu.sync_copy(ones_hbm, ones_v)
        pltpu.sync_copy(edges_hbm.at[1, wid], idx_v)
        plsc.subcore_barrier()

        def fire(j, carry):
            pltpu.async_copy(ones_v, deg_sh.at[idx_v.at[j]], dsem, add=True)
            return carry

        lax.fori_loop(0, rows_pt, fire, 0)

        def drain(j, carry):
            pltpu.make_async_copy(ones_v, deg_sh.at[idx_v.at[j]], dsem).wait()
            return carry

        lax.fori_loop(0, rows_pt, drain, 0)
        plsc.subcore_barrier()

        @pl.when(s < N_ZT)
        def _out():
            pltpu.sync_copy(deg_sh.at[pl.ds(s * rpt, rpt)],
                            out_hbm.at[c, pl.ds(s * rpt, rpt)])

    return deg_kernel


def _make_sc_scatter(n_nodes, n_acc, rows_pt, d):
    n_grp = rows_pt // NB
    zpt = n_acc // N_ZT
    rpt = n_nodes // N_ZT
    ns = 2 if d >= 64 else 3

    @functools.partial(
        pl.kernel,
        out_type=jax.ShapeDtypeStruct((N_CORES, n_nodes, d), jnp.float32),
        mesh=_mesh(),
        compiler_params=pltpu.CompilerParams(use_tc_tiling_on_sc=False),
        scratch_types=[
            pltpu.VMEM((rows_pt, KB), jnp.int32),
            pltpu.VMEM((rows_pt, KB), jnp.int32),
            pltpu.VMEM((ns, NB, KB, d), jnp.float32),
            pltpu.VMEM_SHARED((n_acc, d), jnp.float32),
            pltpu.SemaphoreType.DMA((ns, NB)),
            pltpu.SemaphoreType.DMA((ns, NB)),
        ],
    )
    def scatter_kernel(edges_hbm, feat_hbm, zeros_hbm, out_hbm,
                       sidx_v, didx_v, rows_v, acc_sh, gsem, ssem):
        c = lax.axis_index("c")
        s = lax.axis_index("s")
        wid = c * N_SUBCORES + s

        @pl.when(s < N_ZT)
        def _zero():
            pltpu.sync_copy(zeros_hbm.at[pl.ds(s * zpt, zpt)],
                            acc_sh.at[pl.ds(s * zpt, zpt)])

        pltpu.sync_copy(edges_hbm.at[0, wid], sidx_v)
        pltpu.sync_copy(edges_hbm.at[1, wid], didx_v)
        plsc.subcore_barrier()

        def g_issue(p, b, j):
            pltpu.async_copy(feat_hbm.at[sidx_v.at[j]], rows_v.at[p, b],
                             gsem.at[p, b])

        def g_wait(p, b, j):
            pltpu.make_async_copy(feat_hbm.at[sidx_v.at[j]], rows_v.at[p, b],
                                  gsem.at[p, b]).wait()

        def s_issue(p, b, j):
            pltpu.async_copy(rows_v.at[p, b], acc_sh.at[didx_v.at[j]],
                             ssem.at[p, b], add=True)

        def s_wait(p, b, j):
            pltpu.make_async_copy(rows_v.at[p, b], acc_sh.at[didx_v.at[j]],
                                  ssem.at[p, b]).wait()

        def step(g, p, do_swait=True, do_gissue=True):
            base = g * NB
            for b in range(NB):
                g_wait(p, b, base + b)
                s_issue(p, b, base + b)
            q = (p + ns - 1) % ns
            for b in range(NB):
                if do_swait:
                    s_wait(q, b, base - NB + b)
                if do_gissue:
                    g_issue(q, b, base + (ns - 1) * NB + b)

        for g0 in range(ns - 1):
            for b in range(NB):
                g_issue(g0, b, g0 * NB + b)
        step(0, 0, do_swait=False)

        n_steady = n_grp - ns
        n_iter = n_steady // ns

        def outer(gi, carry):
            g = ns * gi + 1
            for r in range(ns):
                step(g + r, (1 + r) % ns)
            return carry

        lax.fori_loop(0, n_iter, outer, 0)
        for g in range(n_iter * ns + 1, n_grp):
            step(g, g % ns, do_gissue=(g + ns - 1 < n_grp))
        for b in range(NB):
            s_wait((n_grp - 1) % ns, b, (n_grp - 1) * NB + b)
        plsc.subcore_barrier()

        @pl.when(s < N_ZT)
        def _out():
            pltpu.sync_copy(acc_sh.at[pl.ds(s * rpt, rpt)],
                            out_hbm.at[c, pl.ds(s * rpt, rpt)])

    return scatter_kernel


def _tc_layer0(x, w0, b0, degp, *, rows_blk=2000):
    n, d_in = x.shape
    d_hid = w0.shape[1]
    grid = (n // rows_blk,)

    def body(x_ref, w0_ref, b0_ref, degp_ref, hs_ref, dinv_ref):
        deg = degp_ref[0, :, 0:1] + degp_ref[1, :, 0:1] + 1.0
        dinv = lax.rsqrt(deg)
        h = jnp.dot(x_ref[...], w0_ref[...],
                    preferred_element_type=jnp.float32) + b0_ref[...]
        hs_ref[...] = h * dinv
        dinv_ref[...] = dinv

    return pl.pallas_call(
        body,
        grid=grid,
        in_specs=[
            pl.BlockSpec((rows_blk, d_in), lambda i: (i, 0)),
            pl.BlockSpec((d_in, d_hid), lambda i: (0, 0)),
            pl.BlockSpec((1, d_hid), lambda i: (0, 0)),
            pl.BlockSpec((N_CORES, rows_blk, 8), lambda i: (0, i, 0)),
        ],
        out_specs=[
            pl.BlockSpec((rows_blk, d_hid), lambda i: (i, 0)),
            pl.BlockSpec((rows_blk, 1), lambda i: (i, 0)),
        ],
        out_shape=[
            jax.ShapeDtypeStruct((n, d_hid), jnp.float32),
            jax.ShapeDtypeStruct((n, 1), jnp.float32),
        ],
    )(x, w0, b0, degp)


def _tc_layer1(accp, hs, dinv, w1, b1, *, rows_blk=2000):
    n, d_hid = hs.shape
    d_out = w1.shape[1]
    grid = (n // rows_blk,)

    def body(accp_ref, hs_ref, dinv_ref, w1_ref, b1_ref, os_ref):
        dinv = dinv_ref[...]
        sm = dinv * (accp_ref[0] + accp_ref[1] + hs_ref[...])
        h1 = jnp.maximum(sm, 0.0)
        os_ref[...] = (jnp.dot(h1, w1_ref[...],
                               preferred_element_type=jnp.float32)
                       + b1_ref[...]) * dinv

    return pl.pallas_call(
        body,
        grid=grid,
        in_specs=[
            pl.BlockSpec((N_CORES, rows_blk, d_hid), lambda i: (0, i, 0)),
            pl.BlockSpec((rows_blk, d_hid), lambda i: (i, 0)),
            pl.BlockSpec((rows_blk, 1), lambda i: (i, 0)),
            pl.BlockSpec((d_hid, d_out), lambda i: (0, 0)),
            pl.BlockSpec((1, d_out), lambda i: (0, 0)),
        ],
        out_specs=pl.BlockSpec((rows_blk, d_out), lambda i: (i, 0)),
        out_shape=jax.ShapeDtypeStruct((n, d_out), jnp.float32),
    )(accp, hs, dinv, w1, b1)


def _tc_final(accp2, os_, dinv, *, rows_blk=2000):
    n, d_out = os_.shape
    grid = (n // rows_blk,)

    def body(accp_ref, os_ref, dinv_ref, o_ref):
        o_ref[...] = dinv_ref[...] * (accp_ref[0] + accp_ref[1] + os_ref[...])

    return pl.pallas_call(
        body,
        grid=grid,
        in_specs=[
            pl.BlockSpec((N_CORES, rows_blk, d_out), lambda i: (0, i, 0)),
            pl.BlockSpec((rows_blk, d_out), lambda i: (i, 0)),
            pl.BlockSpec((rows_blk, 1), lambda i: (i, 0)),
        ],
        out_specs=pl.BlockSpec((rows_blk, d_out), lambda i: (i, 0)),
        out_shape=jax.ShapeDtypeStruct((n, d_out), jnp.float32),
    )(accp2, os_, dinv)


def kernel(X, edge_index, W0, b0, W1, b1):
    n, d_in = X.shape
    d_hid = W0.shape[1]
    d_out = W1.shape[1]
    e = edge_index.shape[1]

    chunk = NW * KB * NB
    e_pad = ((e + chunk - 1) // chunk) * chunk
    rows_pt = e_pad // (NW * KB)
    pad_n = e_pad - e
    n_acc = ((n + 1 + 8 * N_ZT - 1) // (8 * N_ZT)) * (8 * N_ZT)
    trash = n + jnp.arange(pad_n, dtype=jnp.int32) % jnp.int32(n_acc - n)
    src_p = jnp.concatenate([edge_index[0], jnp.zeros((pad_n,), jnp.int32)])
    dst_p = jnp.concatenate([edge_index[1], trash])
    edges4d = jnp.stack([src_p, dst_p]).reshape(2, NW, rows_pt, KB)
    b0r = b0.reshape(1, d_hid)
    b1r = b1.reshape(1, d_out)

    zeros_deg = jnp.zeros((n_acc, 8), jnp.float32)
    ones8 = jnp.ones((KB, 8), jnp.float32)
    zeros_hid = jnp.zeros((n_acc, d_hid), jnp.float32)
    zeros_out = jnp.zeros((n_acc, d_out), jnp.float32)

    degp = _make_sc_degree(n, n_acc, rows_pt)(edges4d, zeros_deg, ones8)
    hs, dinv = _tc_layer0(X, W0, b0r, degp)
    accp = _make_sc_scatter(n, n_acc, rows_pt, d_hid)(edges4d, hs, zeros_hid)
    os_ = _tc_layer1(accp, hs, dinv, W1, b1r)
    accp2 = _make_sc_scatter(n, n_acc, rows_pt, d_out)(edges4d, os_, zeros_out)
    return _tc_final(accp2, os_, dinv)


# speedup vs baseline: 2.6338x; 1.1287x over previous
import functools

import jax
import jax.numpy as jnp
from jax import lax
from jax.experimental import pallas as pl
from jax.experimental.pallas import tpu as pltpu
from jax.experimental.pallas import tpu_sc as plsc

N_CORES = 2
N_SUBCORES = 16
NW = N_CORES * N_SUBCORES
KB = 80
NB = 5
N_ZT = 10


def _mesh():
    return plsc.VectorSubcoreMesh(core_axis_name="c", subcore_axis_name="s")


def _make_sc_degree(n_nodes, n_acc, rows_pt):
    zpt = n_acc // N_ZT
    rpt = n_nodes // N_ZT

    @functools.partial(
        pl.kernel,
        out_type=jax.ShapeDtypeStruct((N_CORES, n_nodes, 8), jnp.float32),
        mesh=_mesh(),
        compiler_params=pltpu.CompilerParams(use_tc_tiling_on_sc=False),
        scratch_types=[
            pltpu.VMEM((rows_pt, KB), jnp.int32),
            pltpu.VMEM((KB, 8), jnp.float32),
            pltpu.VMEM_SHARED((n_acc, 8), jnp.float32),
            pltpu.SemaphoreType.DMA,
        ],
    )
    def deg_kernel(edges_hbm, zeros_hbm, ones_hbm, out_hbm,
                   idx_v, ones_v, deg_sh, dsem):
        c = lax.axis_index("c")
        s = lax.axis_index("s")
        wid = c * N_SUBCORES + s

        @pl.when(s < N_ZT)
        def _zero():
            pltpu.sync_copy(zeros_hbm.at[pl.ds(s * zpt, zpt)],
                            deg_sh.at[pl.ds(s * zpt, zpt)])

        pltpu.sync_copy(ones_hbm, ones_v)
        pltpu.sync_copy(edges_hbm.at[1, wid], idx_v)
        plsc.subcore_barrier()

        def fire(j, carry):
            pltpu.async_copy(ones_v, deg_sh.at[idx_v.at[j]], dsem, add=True)
            return carry

        lax.fori_loop(0, rows_pt, fire, 0)

        def drain(j, carry):
            pltpu.make_async_copy(ones_v, deg_sh.at[idx_v.at[j]], dsem).wait()
            return carry

        lax.fori_loop(0, rows_pt, drain, 0)
        plsc.subcore_barrier()

        @pl.when(s < N_ZT)
        def _out():
            pltpu.sync_copy(deg_sh.at[pl.ds(s * rpt, rpt)],
                            out_hbm.at[c, pl.ds(s * rpt, rpt)])

    return deg_kernel


def _make_sc_scatter(n_nodes, n_acc, rows_pt, d, dtype=jnp.float32):
    n_grp = rows_pt // NB
    zpt = n_acc // N_ZT
    rpt = n_nodes // N_ZT
    ns = 2 if d >= 64 else 3

    @functools.partial(
        pl.kernel,
        out_type=jax.ShapeDtypeStruct((N_CORES, n_nodes, d), dtype),
        mesh=_mesh(),
        compiler_params=pltpu.CompilerParams(use_tc_tiling_on_sc=False),
        scratch_types=[
            pltpu.VMEM((rows_pt, KB), jnp.int32),
            pltpu.VMEM((rows_pt, KB), jnp.int32),
            pltpu.VMEM((ns, NB, KB, d), dtype),
            pltpu.VMEM_SHARED((n_acc, d), dtype),
            pltpu.SemaphoreType.DMA((ns, NB)),
            pltpu.SemaphoreType.DMA((ns, NB)),
        ],
    )
    def scatter_kernel(edges_hbm, feat_hbm, zeros_hbm, out_hbm,
                       sidx_v, didx_v, rows_v, acc_sh, gsem, ssem):
        c = lax.axis_index("c")
        s = lax.axis_index("s")
        wid = c * N_SUBCORES + s

        @pl.when(s < N_ZT)
        def _zero():
            pltpu.sync_copy(zeros_hbm.at[pl.ds(s * zpt, zpt)],
                            acc_sh.at[pl.ds(s * zpt, zpt)])

        pltpu.sync_copy(edges_hbm.at[0, wid], sidx_v)
        pltpu.sync_copy(edges_hbm.at[1, wid], didx_v)
        plsc.subcore_barrier()

        def g_issue(p, b, j):
            pltpu.async_copy(feat_hbm.at[sidx_v.at[j]], rows_v.at[p, b],
                             gsem.at[p, b])

        def g_wait(p, b, j):
            pltpu.make_async_copy(feat_hbm.at[sidx_v.at[j]], rows_v.at[p, b],
                                  gsem.at[p, b]).wait()

        def s_issue(p, b, j):
            pltpu.async_copy(rows_v.at[p, b], acc_sh.at[didx_v.at[j]],
                             ssem.at[p, b], add=True)

        def s_wait(p, b, j):
            pltpu.make_async_copy(rows_v.at[p, b], acc_sh.at[didx_v.at[j]],
                                  ssem.at[p, b]).wait()

        def step(g, p, do_swait=True, do_gissue=True):
            base = g * NB
            for b in range(NB):
                g_wait(p, b, base + b)
                s_issue(p, b, base + b)
            q = (p + ns - 1) % ns
            for b in range(NB):
                if do_swait:
                    s_wait(q, b, base - NB + b)
                if do_gissue:
                    g_issue(q, b, base + (ns - 1) * NB + b)

        for g0 in range(ns - 1):
            for b in range(NB):
                g_issue(g0, b, g0 * NB + b)
        step(0, 0, do_swait=False)

        n_steady = n_grp - ns
        n_iter = n_steady // ns

        def outer(gi, carry):
            g = ns * gi + 1
            for r in range(ns):
                step(g + r, (1 + r) % ns)
            return carry

        lax.fori_loop(0, n_iter, outer, 0)
        for g in range(n_iter * ns + 1, n_grp):
            step(g, g % ns, do_gissue=(g + ns - 1 < n_grp))
        for b in range(NB):
            s_wait((n_grp - 1) % ns, b, (n_grp - 1) * NB + b)
        plsc.subcore_barrier()

        @pl.when(s < N_ZT)
        def _out():
            pltpu.sync_copy(acc_sh.at[pl.ds(s * rpt, rpt)],
                            out_hbm.at[c, pl.ds(s * rpt, rpt)])

    return scatter_kernel


def _tc_layer0(x, w0, b0, degp, *, rows_blk=2000):
    n, d_in = x.shape
    d_hid = w0.shape[1]
    grid = (n // rows_blk,)

    def body(x_ref, w0_ref, b0_ref, degp_ref, hs_ref, hsb_ref, dinv_ref):
        deg = degp_ref[0, :, 0:1] + degp_ref[1, :, 0:1] + 1.0
        dinv = lax.rsqrt(deg)
        h = jnp.dot(x_ref[...], w0_ref[...],
                    preferred_element_type=jnp.float32) + b0_ref[...]
        hs = h * dinv
        hs_ref[...] = hs
        hsb_ref[...] = hs.astype(jnp.bfloat16)
        dinv_ref[...] = dinv

    return pl.pallas_call(
        body,
        grid=grid,
        in_specs=[
            pl.BlockSpec((rows_blk, d_in), lambda i: (i, 0)),
            pl.BlockSpec((d_in, d_hid), lambda i: (0, 0)),
            pl.BlockSpec((1, d_hid), lambda i: (0, 0)),
            pl.BlockSpec((N_CORES, rows_blk, 8), lambda i: (0, i, 0)),
        ],
        out_specs=[
            pl.BlockSpec((rows_blk, d_hid), lambda i: (i, 0)),
            pl.BlockSpec((rows_blk, d_hid), lambda i: (i, 0)),
            pl.BlockSpec((rows_blk, 1), lambda i: (i, 0)),
        ],
        out_shape=[
            jax.ShapeDtypeStruct((n, d_hid), jnp.float32),
            jax.ShapeDtypeStruct((n, d_hid), jnp.bfloat16),
            jax.ShapeDtypeStruct((n, 1), jnp.float32),
        ],
    )(x, w0, b0, degp)


def _tc_layer1(accp, hs, dinv, w1, b1, *, rows_blk=2000):
    n, d_hid = hs.shape
    d_out = w1.shape[1]
    grid = (n // rows_blk,)

    def body(accp_ref, hs_ref, dinv_ref, w1_ref, b1_ref, os_ref):
        dinv = dinv_ref[...]
        acc = (accp_ref[0] + accp_ref[1]).astype(jnp.float32)
        sm = dinv * (acc + hs_ref[...])
        h1 = jnp.maximum(sm, 0.0)
        os_ref[...] = (jnp.dot(h1, w1_ref[...],
                               preferred_element_type=jnp.float32)
                       + b1_ref[...]) * dinv

    return pl.pallas_call(
        body,
        grid=grid,
        in_specs=[
            pl.BlockSpec((N_CORES, rows_blk, d_hid), lambda i: (0, i, 0)),
            pl.BlockSpec((rows_blk, d_hid), lambda i: (i, 0)),
            pl.BlockSpec((rows_blk, 1), lambda i: (i, 0)),
            pl.BlockSpec((d_hid, d_out), lambda i: (0, 0)),
            pl.BlockSpec((1, d_out), lambda i: (0, 0)),
        ],
        out_specs=pl.BlockSpec((rows_blk, d_out), lambda i: (i, 0)),
        out_shape=jax.ShapeDtypeStruct((n, d_out), jnp.float32),
    )(accp, hs, dinv, w1, b1)


def _tc_final(accp2, os_, dinv, *, rows_blk=2000):
    n, d_out = os_.shape
    grid = (n // rows_blk,)

    def body(accp_ref, os_ref, dinv_ref, o_ref):
        o_ref[...] = dinv_ref[...] * (accp_ref[0] + accp_ref[1] + os_ref[...])

    return pl.pallas_call(
        body,
        grid=grid,
        in_specs=[
            pl.BlockSpec((N_CORES, rows_blk, d_out), lambda i: (0, i, 0)),
            pl.BlockSpec((rows_blk, d_out), lambda i: (i, 0)),
            pl.BlockSpec((rows_blk, 1), lambda i: (i, 0)),
        ],
        out_specs=pl.BlockSpec((rows_blk, d_out), lambda i: (i, 0)),
        out_shape=jax.ShapeDtypeStruct((n, d_out), jnp.float32),
    )(accp2, os_, dinv)


def kernel(X, edge_index, W0, b0, W1, b1):
    n, d_in = X.shape
    d_hid = W0.shape[1]
    d_out = W1.shape[1]
    e = edge_index.shape[1]

    chunk = NW * KB * NB
    e_pad = ((e + chunk - 1) // chunk) * chunk
    rows_pt = e_pad // (NW * KB)
    pad_n = e_pad - e
    n_acc = ((n + 1 + 8 * N_ZT - 1) // (8 * N_ZT)) * (8 * N_ZT)
    trash = n + jnp.arange(pad_n, dtype=jnp.int32) % jnp.int32(n_acc - n)
    src_p = jnp.concatenate([edge_index[0], jnp.zeros((pad_n,), jnp.int32)])
    dst_p = jnp.concatenate([edge_index[1], trash])
    edges4d = jnp.stack([src_p, dst_p]).reshape(2, NW, rows_pt, KB)
    b0r = b0.reshape(1, d_hid)
    b1r = b1.reshape(1, d_out)

    zeros_deg = jnp.zeros((n_acc, 8), jnp.float32)
    ones8 = jnp.ones((KB, 8), jnp.float32)
    zeros_hid = jnp.zeros((n_acc, d_hid), jnp.bfloat16)
    zeros_out = jnp.zeros((n_acc, d_out), jnp.float32)

    degp = _make_sc_degree(n, n_acc, rows_pt)(edges4d, zeros_deg, ones8)
    hs, hsb, dinv = _tc_layer0(X, W0, b0r, degp)
    accp = _make_sc_scatter(n, n_acc, rows_pt, d_hid,
                            dtype=jnp.bfloat16)(edges4d, hsb, zeros_hid)
    os_ = _tc_layer1(accp, hs, dinv, W1, b1r)
    accp2 = _make_sc_scatter(n, n_acc, rows_pt, d_out)(edges4d, os_, zeros_out)
    return _tc_final(accp2, os_, dinv)


# speedup vs baseline: 2.6996x; 1.0250x over previous
import functools

import jax
import jax.numpy as jnp
from jax import lax
from jax.experimental import pallas as pl
from jax.experimental.pallas import tpu as pltpu
from jax.experimental.pallas import tpu_sc as plsc

N_CORES = 2
N_SUBCORES = 16
NW = N_CORES * N_SUBCORES
KB = 80
NB = 5
N_ZT = 10


def _mesh():
    return plsc.VectorSubcoreMesh(core_axis_name="c", subcore_axis_name="s")


def _make_sc_degree(n_nodes, n_acc, rows_pt):
    zpt = n_acc // N_ZT
    rpt = n_nodes // N_ZT

    @functools.partial(
        pl.kernel,
        out_type=jax.ShapeDtypeStruct((N_CORES, n_nodes, 8), jnp.float32),
        mesh=_mesh(),
        compiler_params=pltpu.CompilerParams(use_tc_tiling_on_sc=False),
        scratch_types=[
            pltpu.VMEM((rows_pt, KB), jnp.int32),
            pltpu.VMEM((KB, 8), jnp.float32),
            pltpu.VMEM_SHARED((n_acc, 8), jnp.float32),
            pltpu.SemaphoreType.DMA,
        ],
    )
    def deg_kernel(edges_hbm, zeros_hbm, ones_hbm, out_hbm,
                   idx_v, ones_v, deg_sh, dsem):
        c = lax.axis_index("c")
        s = lax.axis_index("s")
        wid = c * N_SUBCORES + s

        @pl.when(s < N_ZT)
        def _zero():
            pltpu.sync_copy(zeros_hbm.at[pl.ds(s * zpt, zpt)],
                            deg_sh.at[pl.ds(s * zpt, zpt)])

        pltpu.sync_copy(ones_hbm, ones_v)
        pltpu.sync_copy(edges_hbm.at[1, wid], idx_v)
        plsc.subcore_barrier()

        def fire(j, carry):
            pltpu.async_copy(ones_v, deg_sh.at[idx_v.at[j]], dsem, add=True)
            return carry

        lax.fori_loop(0, rows_pt, fire, 0)

        def drain(j, carry):
            pltpu.make_async_copy(ones_v, deg_sh.at[idx_v.at[j]], dsem).wait()
            return carry

        lax.fori_loop(0, rows_pt, drain, 0)
        plsc.subcore_barrier()

        @pl.when(s < N_ZT)
        def _out():
            pltpu.sync_copy(deg_sh.at[pl.ds(s * rpt, rpt)],
                            out_hbm.at[c, pl.ds(s * rpt, rpt)])

    return deg_kernel


def _make_sc_scatter(n_nodes, n_acc, rows_pt, d, dtype=jnp.float32):
    n_grp = rows_pt // NB
    zpt = n_acc // N_ZT
    rpt = n_nodes // N_ZT
    ns = 3

    @functools.partial(
        pl.kernel,
        out_type=jax.ShapeDtypeStruct((N_CORES, n_nodes, d), dtype),
        mesh=_mesh(),
        compiler_params=pltpu.CompilerParams(use_tc_tiling_on_sc=False),
        scratch_types=[
            pltpu.VMEM((rows_pt, KB), jnp.int32),
            pltpu.VMEM((rows_pt, KB), jnp.int32),
            pltpu.VMEM((ns, NB, KB, d), dtype),
            pltpu.VMEM_SHARED((n_acc, d), dtype),
            pltpu.SemaphoreType.DMA((ns, NB)),
            pltpu.SemaphoreType.DMA((ns, NB)),
        ],
    )
    def scatter_kernel(edges_hbm, feat_hbm, zeros_hbm, out_hbm,
                       sidx_v, didx_v, rows_v, acc_sh, gsem, ssem):
        c = lax.axis_index("c")
        s = lax.axis_index("s")
        wid = c * N_SUBCORES + s

        @pl.when(s < N_ZT)
        def _zero():
            pltpu.sync_copy(zeros_hbm.at[pl.ds(s * zpt, zpt)],
                            acc_sh.at[pl.ds(s * zpt, zpt)])

        pltpu.sync_copy(edges_hbm.at[0, wid], sidx_v)
        pltpu.sync_copy(edges_hbm.at[1, wid], didx_v)
        plsc.subcore_barrier()

        def g_issue(p, b, j):
            pltpu.async_copy(feat_hbm.at[sidx_v.at[j]], rows_v.at[p, b],
                             gsem.at[p, b])

        def g_wait(p, b, j):
            pltpu.make_async_copy(feat_hbm.at[sidx_v.at[j]], rows_v.at[p, b],
                                  gsem.at[p, b]).wait()

        def s_issue(p, b, j):
            pltpu.async_copy(rows_v.at[p, b], acc_sh.at[didx_v.at[j]],
                             ssem.at[p, b], add=True)

        def s_wait(p, b, j):
            pltpu.make_async_copy(rows_v.at[p, b], acc_sh.at[didx_v.at[j]],
                                  ssem.at[p, b]).wait()

        def step(g, p, do_swait=True, do_gissue=True):
            base = g * NB
            for b in range(NB):
                g_wait(p, b, base + b)
                s_issue(p, b, base + b)
            q = (p + ns - 1) % ns
            for b in range(NB):
                if do_swait:
                    s_wait(q, b, base - NB + b)
                if do_gissue:
                    g_issue(q, b, base + (ns - 1) * NB + b)

        for g0 in range(ns - 1):
            for b in range(NB):
                g_issue(g0, b, g0 * NB + b)
        step(0, 0, do_swait=False)

        n_steady = n_grp - ns
        n_iter = n_steady // ns

        def outer(gi, carry):
            g = ns * gi + 1
            for r in range(ns):
                step(g + r, (1 + r) % ns)
            return carry

        lax.fori_loop(0, n_iter, outer, 0)
        for g in range(n_iter * ns + 1, n_grp):
            step(g, g % ns, do_gissue=(g + ns - 1 < n_grp))
        for b in range(NB):
            s_wait((n_grp - 1) % ns, b, (n_grp - 1) * NB + b)
        plsc.subcore_barrier()

        @pl.when(s < N_ZT)
        def _out():
            pltpu.sync_copy(acc_sh.at[pl.ds(s * rpt, rpt)],
                            out_hbm.at[c, pl.ds(s * rpt, rpt)])

    return scatter_kernel


def _tc_layer0(x, w0, b0, degp, *, rows_blk=2000):
    n, d_in = x.shape
    d_hid = w0.shape[1]
    grid = (n // rows_blk,)

    def body(x_ref, w0_ref, b0_ref, degp_ref, hs_ref, hsb_ref, dinv_ref):
        deg = degp_ref[0, :, 0:1] + degp_ref[1, :, 0:1] + 1.0
        dinv = lax.rsqrt(deg)
        h = jnp.dot(x_ref[...].astype(jnp.bfloat16),
                    w0_ref[...].astype(jnp.bfloat16),
                    preferred_element_type=jnp.float32) + b0_ref[...]
        hs = h * dinv
        hs_ref[...] = hs
        hsb_ref[...] = hs.astype(jnp.bfloat16)
        dinv_ref[...] = dinv

    return pl.pallas_call(
        body,
        grid=grid,
        in_specs=[
            pl.BlockSpec((rows_blk, d_in), lambda i: (i, 0)),
            pl.BlockSpec((d_in, d_hid), lambda i: (0, 0)),
            pl.BlockSpec((1, d_hid), lambda i: (0, 0)),
            pl.BlockSpec((N_CORES, rows_blk, 8), lambda i: (0, i, 0)),
        ],
        out_specs=[
            pl.BlockSpec((rows_blk, d_hid), lambda i: (i, 0)),
            pl.BlockSpec((rows_blk, d_hid), lambda i: (i, 0)),
            pl.BlockSpec((rows_blk, 1), lambda i: (i, 0)),
        ],
        out_shape=[
            jax.ShapeDtypeStruct((n, d_hid), jnp.float32),
            jax.ShapeDtypeStruct((n, d_hid), jnp.bfloat16),
            jax.ShapeDtypeStruct((n, 1), jnp.float32),
        ],
    )(x, w0, b0, degp)


def _tc_layer1(accp, hs, dinv, w1, b1, *, rows_blk=2000):
    n, d_hid = hs.shape
    d_out = w1.shape[1]
    grid = (n // rows_blk,)

    def body(accp_ref, hs_ref, dinv_ref, w1_ref, b1_ref, os_ref):
        dinv = dinv_ref[...]
        acc = (accp_ref[0] + accp_ref[1]).astype(jnp.float32)
        sm = dinv * (acc + hs_ref[...])
        h1 = jnp.maximum(sm, 0.0)
        os_ref[...] = (jnp.dot(h1.astype(jnp.bfloat16),
                               w1_ref[...].astype(jnp.bfloat16),
                               preferred_element_type=jnp.float32)
                       + b1_ref[...]) * dinv

    return pl.pallas_call(
        body,
        grid=grid,
        in_specs=[
            pl.BlockSpec((N_CORES, rows_blk, d_hid), lambda i: (0, i, 0)),
            pl.BlockSpec((rows_blk, d_hid), lambda i: (i, 0)),
            pl.BlockSpec((rows_blk, 1), lambda i: (i, 0)),
            pl.BlockSpec((d_hid, d_out), lambda i: (0, 0)),
            pl.BlockSpec((1, d_out), lambda i: (0, 0)),
        ],
        out_specs=pl.BlockSpec((rows_blk, d_out), lambda i: (i, 0)),
        out_shape=jax.ShapeDtypeStruct((n, d_out), jnp.float32),
    )(accp, hs, dinv, w1, b1)


def _tc_final(accp2, os_, dinv, *, rows_blk=2000):
    n, d_out = os_.shape
    grid = (n // rows_blk,)

    def body(accp_ref, os_ref, dinv_ref, o_ref):
        o_ref[...] = dinv_ref[...] * (accp_ref[0] + accp_ref[1] + os_ref[...])

    return pl.pallas_call(
        body,
        grid=grid,
        in_specs=[
            pl.BlockSpec((N_CORES, rows_blk, d_out), lambda i: (0, i, 0)),
            pl.BlockSpec((rows_blk, d_out), lambda i: (i, 0)),
            pl.BlockSpec((rows_blk, 1), lambda i: (i, 0)),
        ],
        out_specs=pl.BlockSpec((rows_blk, d_out), lambda i: (i, 0)),
        out_shape=jax.ShapeDtypeStruct((n, d_out), jnp.float32),
    )(accp2, os_, dinv)


def kernel(X, edge_index, W0, b0, W1, b1):
    n, d_in = X.shape
    d_hid = W0.shape[1]
    d_out = W1.shape[1]
    e = edge_index.shape[1]

    chunk = NW * KB * NB
    e_pad = ((e + chunk - 1) // chunk) * chunk
    rows_pt = e_pad // (NW * KB)
    pad_n = e_pad - e
    n_acc = ((n + 1 + 8 * N_ZT - 1) // (8 * N_ZT)) * (8 * N_ZT)
    trash = n + jnp.arange(pad_n, dtype=jnp.int32) % jnp.int32(n_acc - n)
    src_p = jnp.concatenate([edge_index[0], jnp.zeros((pad_n,), jnp.int32)])
    dst_p = jnp.concatenate([edge_index[1], trash])
    edges4d = jnp.stack([src_p, dst_p]).reshape(2, NW, rows_pt, KB)
    b0r = b0.reshape(1, d_hid)
    b1r = b1.reshape(1, d_out)

    zeros_deg = jnp.zeros((n_acc, 8), jnp.float32)
    ones8 = jnp.ones((KB, 8), jnp.float32)
    zeros_hid = jnp.zeros((n_acc, d_hid), jnp.bfloat16)
    zeros_out = jnp.zeros((n_acc, d_out), jnp.float32)

    degp = _make_sc_degree(n, n_acc, rows_pt)(edges4d, zeros_deg, ones8)
    hs, hsb, dinv = _tc_layer0(X, W0, b0r, degp)
    accp = _make_sc_scatter(n, n_acc, rows_pt, d_hid,
                            dtype=jnp.bfloat16)(edges4d, hsb, zeros_hid)
    os_ = _tc_layer1(accp, hs, dinv, W1, b1r)
    accp2 = _make_sc_scatter(n, n_acc, rows_pt, d_out)(edges4d, os_, zeros_out)
    return _tc_final(accp2, os_, dinv)
